# Initial kernel scaffold; baseline (speedup 1.0000x reference)
import jax
import jax.numpy as jnp
from jax.experimental import pallas as pl


def kernel(x0, x1, down_lap_0, incidence_t_0, W_lap, W_inc):
    raise NotImplementedError("write your pallas kernel here")


# speedup vs baseline: 1.3483x; 1.3483x over previous
import jax
import jax.numpy as jnp
from jax.experimental import pallas as pl

_BF = jnp.bfloat16
_F32 = jnp.float32


def _xw_body(x1_ref, x0_ref, wl_ref, wi_ref, a_ref, c_ref):
    a_ref[...] = jnp.dot(
        x1_ref[...].astype(_BF), wl_ref[...].astype(_BF),
        preferred_element_type=_F32).astype(_BF)
    c_ref[...] = jnp.dot(
        x0_ref[...].astype(_BF), wi_ref[...].astype(_BF),
        preferred_element_type=_F32).astype(_BF)


def _nbr_body(l_ref, it_ref, a_ref, c_ref, b_ref, d_ref, sb_ref, sd_ref):
    b = jnp.dot(l_ref[...].astype(_BF), a_ref[...],
                preferred_element_type=_F32)
    d = jnp.dot(it_ref[...].astype(_BF), c_ref[...],
                preferred_element_type=_F32)
    b_ref[...] = b
    d_ref[...] = d
    c = b.shape[1]
    sb_ref[...] = jnp.sum(b, axis=0).reshape(1, 1, c)
    sd_ref[...] = jnp.sum(d, axis=0).reshape(1, 1, c)


def _agg_body(b_ref, d_ref, sb_ref, sd_ref, o_ref):
    sb = jax.nn.relu(jnp.sum(sb_ref[...], axis=0))
    sd = jax.nn.relu(jnp.sum(sd_ref[...], axis=0))
    b = b_ref[...]
    d = d_ref[...]
    bb = b.astype(_BF).astype(_F32)
    db = d.astype(_BF).astype(_F32)
    sbb = sb.astype(_BF).astype(_F32)
    sdb = sd.astype(_BF).astype(_F32)
    tb = jnp.sum(bb * sbb, axis=1, keepdims=True)
    td = jnp.sum(db * sdb, axis=1, keepdims=True)
    wb = 1.0 / (1.0 + jnp.exp(-tb))
    wd = 1.0 / (1.0 + jnp.exp(-td))
    o_ref[...] = jax.nn.relu((wb * b + wd * d) * 0.5)


def kernel(x0, x1, down_lap_0, incidence_t_0, W_lap, W_inc):
    n1, c = x1.shape
    n0 = x0.shape[0]
    blk = 512
    nblk = n1 // blk

    a, cc = pl.pallas_call(
        _xw_body,
        grid=(nblk,),
        in_specs=[
            pl.BlockSpec((blk, c), lambda i: (i, 0)),
            pl.BlockSpec((blk, c), lambda i: (i, 0)),
            pl.BlockSpec((c, c), lambda i: (0, 0)),
            pl.BlockSpec((c, c), lambda i: (0, 0)),
        ],
        out_specs=[
            pl.BlockSpec((blk, c), lambda i: (i, 0)),
            pl.BlockSpec((blk, c), lambda i: (i, 0)),
        ],
        out_shape=[
            jax.ShapeDtypeStruct((n1, c), _BF),
            jax.ShapeDtypeStruct((n0, c), _BF),
        ],
    )(x1, x0, W_lap, W_inc)

    b, d, sb, sd = pl.pallas_call(
        _nbr_body,
        grid=(nblk,),
        in_specs=[
            pl.BlockSpec((blk, n1), lambda i: (i, 0)),
            pl.BlockSpec((blk, n0), lambda i: (i, 0)),
            pl.BlockSpec((n1, c), lambda i: (0, 0)),
            pl.BlockSpec((n0, c), lambda i: (0, 0)),
        ],
        out_specs=[
            pl.BlockSpec((blk, c), lambda i: (i, 0)),
            pl.BlockSpec((blk, c), lambda i: (i, 0)),
            pl.BlockSpec((1, 1, c), lambda i: (i, 0, 0)),
            pl.BlockSpec((1, 1, c), lambda i: (i, 0, 0)),
        ],
        out_shape=[
            jax.ShapeDtypeStruct((n1, c), _F32),
            jax.ShapeDtypeStruct((n1, c), _F32),
            jax.ShapeDtypeStruct((nblk, 1, c), _F32),
            jax.ShapeDtypeStruct((nblk, 1, c), _F32),
        ],
    )(down_lap_0, incidence_t_0, a, cc)

    x1_new = pl.pallas_call(
        _agg_body,
        grid=(nblk,),
        in_specs=[
            pl.BlockSpec((blk, c), lambda i: (i, 0)),
            pl.BlockSpec((blk, c), lambda i: (i, 0)),
            pl.BlockSpec((nblk, 1, c), lambda i: (0, 0, 0)),
            pl.BlockSpec((nblk, 1, c), lambda i: (0, 0, 0)),
        ],
        out_specs=pl.BlockSpec((blk, c), lambda i: (i, 0)),
        out_shape=jax.ShapeDtypeStruct((n1, c), _F32),
    )(b, d, sb, sd)

    return (x0, x1_new)


# speedup vs baseline: 1.4232x; 1.0555x over previous
import jax
import jax.numpy as jnp
from jax.experimental import pallas as pl
from jax.experimental.pallas import tpu as pltpu

_BF = jnp.bfloat16
_F32 = jnp.float32
_BLK = 256


def _fused_body(x1_ref, x0_ref, wl_ref, wi_ref, l_ref, it_ref, o_ref,
                a_s, c_s, b_s, d_s, sb_s, sd_s):
    p = pl.program_id(0)
    i = pl.program_id(1)
    rows = pl.ds(i * _BLK, _BLK)

    @pl.when(p == 0)
    def _feature_gemms():
        a_s[rows, :] = jnp.dot(
            x1_ref[...].astype(_BF), wl_ref[...].astype(_BF),
            preferred_element_type=_F32).astype(_BF)
        c_s[rows, :] = jnp.dot(
            x0_ref[...].astype(_BF), wi_ref[...].astype(_BF),
            preferred_element_type=_F32).astype(_BF)

    @pl.when(p == 1)
    def _neighborhood_gemms():
        b = jnp.dot(l_ref[...].astype(_BF), a_s[...],
                    preferred_element_type=_F32)
        d = jnp.dot(it_ref[...].astype(_BF), c_s[...],
                    preferred_element_type=_F32)
        b_s[rows, :] = b
        d_s[rows, :] = d
        csb = jnp.sum(b, axis=0, keepdims=True)
        csd = jnp.sum(d, axis=0, keepdims=True)

        @pl.when(i == 0)
        def _():
            sb_s[...] = csb
            sd_s[...] = csd

        @pl.when(i > 0)
        def _():
            sb_s[...] += csb
            sd_s[...] += csd

    @pl.when(p == 2)
    def _aggregate():
        sbb = jax.nn.relu(sb_s[...]).astype(_BF).astype(_F32)
        sdb = jax.nn.relu(sd_s[...]).astype(_BF).astype(_F32)
        b = b_s[rows, :]
        d = d_s[rows, :]
        bb = b.astype(_BF).astype(_F32)
        db = d.astype(_BF).astype(_F32)
        tb = jnp.sum(bb * sbb, axis=1, keepdims=True)
        td = jnp.sum(db * sdb, axis=1, keepdims=True)
        wb = 1.0 / (1.0 + jnp.exp(-tb))
        wd = 1.0 / (1.0 + jnp.exp(-td))
        o_ref[...] = jax.nn.relu((wb * b + wd * d) * 0.5)


def kernel(x0, x1, down_lap_0, incidence_t_0, W_lap, W_inc):
    n1, c = x1.shape
    n0 = x0.shape[0]
    nblk = n1 // _BLK
    last = nblk - 1

    x1_new = pl.pallas_call(
        _fused_body,
        grid=(3, nblk),
        in_specs=[
            pl.BlockSpec((_BLK, c), lambda p, i: (jnp.where(p == 0, i, last), 0)),
            pl.BlockSpec((_BLK, c), lambda p, i: (jnp.where(p == 0, i, last), 0)),
            pl.BlockSpec((c, c), lambda p, i: (0, 0)),
            pl.BlockSpec((c, c), lambda p, i: (0, 0)),
            pl.BlockSpec((_BLK, n1),
                         lambda p, i: (jnp.where(p == 1, i,
                                                 jnp.where(p == 0, 0, last)), 0)),
            pl.BlockSpec((_BLK, n0),
                         lambda p, i: (jnp.where(p == 1, i,
                                                 jnp.where(p == 0, 0, last)), 0)),
        ],
        out_specs=pl.BlockSpec((_BLK, c), lambda p, i: (jnp.where(p == 2, i, 0), 0)),
        out_shape=jax.ShapeDtypeStruct((n1, c), _F32),
        scratch_shapes=[
            pltpu.VMEM((n1, c), _BF),
            pltpu.VMEM((n0, c), _BF),
            pltpu.VMEM((n1, c), _F32),
            pltpu.VMEM((n1, c), _F32),
            pltpu.VMEM((1, c), _F32),
            pltpu.VMEM((1, c), _F32),
        ],
    )(x1, x0, W_lap, W_inc, down_lap_0, incidence_t_0)

    return (x0, x1_new)


# speedup vs baseline: 1.5731x; 1.1053x over previous
import jax
import jax.numpy as jnp
from jax.experimental import pallas as pl
from jax.experimental.pallas import tpu as pltpu

_BF = jnp.bfloat16
_F32 = jnp.float32
_BLK = 512


def _fused_body(x1_ref, x0_ref, wl_ref, wi_ref, l_ref, it_ref, o_ref,
                a_s, c_s, b_s, d_s, sb_s, sd_s):
    p = pl.program_id(0)
    i = pl.program_id(1)
    rows = pl.ds(i * _BLK, _BLK)

    @pl.when(p == 0)
    def _feature_gemms():
        a_s[rows, :] = jnp.dot(
            x1_ref[...].astype(_BF), wl_ref[...].astype(_BF),
            preferred_element_type=_F32).astype(_BF)
        c_s[rows, :] = jnp.dot(
            x0_ref[...].astype(_BF), wi_ref[...].astype(_BF),
            preferred_element_type=_F32).astype(_BF)

    @pl.when(p == 1)
    def _neighborhood_gemms():
        b = jnp.dot(l_ref[...].astype(_BF), a_s[...],
                    preferred_element_type=_F32)
        d = jnp.dot(it_ref[...].astype(_BF), c_s[...],
                    preferred_element_type=_F32)
        b_s[rows, :] = b.astype(_BF)
        d_s[rows, :] = d.astype(_BF)
        csb = jnp.sum(b, axis=0, keepdims=True)
        csd = jnp.sum(d, axis=0, keepdims=True)

        @pl.when(i == 0)
        def _():
            sb_s[...] = csb
            sd_s[...] = csd

        @pl.when(i > 0)
        def _():
            sb_s[...] += csb
            sd_s[...] += csd

    @pl.when(p == 2)
    def _aggregate():
        sbb = jax.nn.relu(sb_s[...]).astype(_BF).astype(_F32)
        sdb = jax.nn.relu(sd_s[...]).astype(_BF).astype(_F32)
        b = b_s[rows, :].astype(_F32)
        d = d_s[rows, :].astype(_F32)
        bb = b
        db = d
        tb = jnp.sum(bb * sbb, axis=1, keepdims=True)
        td = jnp.sum(db * sdb, axis=1, keepdims=True)
        wb = 1.0 / (1.0 + jnp.exp(-tb))
        wd = 1.0 / (1.0 + jnp.exp(-td))
        o_ref[...] = jax.nn.relu((wb * b + wd * d) * 0.5)


def kernel(x0, x1, down_lap_0, incidence_t_0, W_lap, W_inc):
    n1, c = x1.shape
    n0 = x0.shape[0]
    nblk = n1 // _BLK
    last = nblk - 1

    x1_new = pl.pallas_call(
        _fused_body,
        grid=(3, nblk),
        in_specs=[
            pl.BlockSpec((_BLK, c), lambda p, i: (jnp.where(p == 0, i, last), 0)),
            pl.BlockSpec((_BLK, c), lambda p, i: (jnp.where(p == 0, i, last), 0)),
            pl.BlockSpec((c, c), lambda p, i: (0, 0)),
            pl.BlockSpec((c, c), lambda p, i: (0, 0)),
            pl.BlockSpec((_BLK, n1),
                         lambda p, i: (jnp.where(p == 1, i,
                                                 jnp.where(p == 0, 0, last)), 0)),
            pl.BlockSpec((_BLK, n0),
                         lambda p, i: (jnp.where(p == 1, i,
                                                 jnp.where(p == 0, 0, last)), 0)),
        ],
        out_specs=pl.BlockSpec((_BLK, c), lambda p, i: (jnp.where(p == 2, i, 0), 0)),
        out_shape=jax.ShapeDtypeStruct((n1, c), _F32),
        scratch_shapes=[
            pltpu.VMEM((n1, c), _BF),
            pltpu.VMEM((n0, c), _BF),
            pltpu.VMEM((n1, c), _BF),
            pltpu.VMEM((n1, c), _BF),
            pltpu.VMEM((1, c), _F32),
            pltpu.VMEM((1, c), _F32),
        ],
    )(x1, x0, W_lap, W_inc, down_lap_0, incidence_t_0)

    return (x0, x1_new)
